# Initial kernel scaffold; baseline (speedup 1.0000x reference)
import jax
import jax.numpy as jnp
from jax.experimental import pallas as pl


def kernel(gnn_emb, edge_index):
    raise NotImplementedError("write your pallas kernel here")


# speedup vs baseline: 5.7842x; 5.7842x over previous
import functools

import jax
import jax.numpy as jnp
from jax import lax
from jax.experimental import pallas as pl
from jax.experimental.pallas import tpu as pltpu
from jax.experimental.pallas import tpu_sc as plsc

D = 128
NV = 10000
NE = 320000

_info = plsc.get_sparse_core_info()
NC, NS, L = _info.num_cores, _info.num_subcores, _info.num_lanes
NW = NC * NS
EPW = NE // NW
C = 400
NCHUNK = EPW // C
G = C // L

_mesh = plsc.VectorSubcoreMesh(core_axis_name="c", subcore_axis_name="s")


@functools.partial(
    pl.kernel,
    mesh=_mesh,
    compiler_params=pltpu.CompilerParams(
        use_tc_tiling_on_sc=False, needs_layout_passes=False
    ),
    out_type=jax.ShapeDtypeStruct((NE,), jnp.float32),
    scratch_types=[
        pltpu.VMEM((C,), jnp.int32),
        pltpu.VMEM((C,), jnp.int32),
        pltpu.VMEM((C, D), jnp.float32),
        pltpu.VMEM((C, D), jnp.float32),
        pltpu.VMEM((C,), jnp.float32),
        pltpu.SemaphoreType.DMA,
        pltpu.SemaphoreType.DMA,
    ],
)
def _score_kernel(table_hbm, ei_hbm, out_hbm, sidx, didx, srows, drows, sc,
                  sem_s, sem_d):
    wid = lax.axis_index("s") * NC + lax.axis_index("c")
    ebase = wid * EPW
    lane = lax.iota(jnp.int32, L)

    def chunk_body(ci, carry):
        base = ebase + ci * C
        pltpu.sync_copy(ei_hbm.at[pl.ds(base, C)], sidx)
        pltpu.sync_copy(ei_hbm.at[pl.ds(NE + base, C)], didx)

        def dbl(i, c):
            sidx[pl.ds(i * L, L)] = sidx[pl.ds(i * L, L)] * 2
            didx[pl.ds(i * L, L)] = didx[pl.ds(i * L, L)] * 2
            return c

        lax.fori_loop(0, C // L, dbl, 0)

        cp_s = pltpu.async_copy(table_hbm.at[sidx], srows, sem_s)
        cp_d = pltpu.async_copy(table_hbm.at[didx], drows, sem_d)
        cp_s.wait()
        cp_d.wait()

        def grp(g, c):
            rows = g * L + lane

            def dstep(d, acc):
                col = (d + lane) & (D - 1)
                s = plsc.load_gather(srows, [rows, col])
                t = plsc.load_gather(drows, [rows, col])
                return acc + s * t

            acc = lax.fori_loop(0, D, dstep, jnp.zeros((L,), jnp.float32),
                                unroll=8)
            sc[pl.ds(g * L, L)] = acc
            return c

        lax.fori_loop(0, G, grp, 0)
        pltpu.sync_copy(sc, out_hbm.at[pl.ds(base, C)])
        return carry

    lax.fori_loop(0, NCHUNK, chunk_body, 0)


def kernel(gnn_emb, edge_index):
    table = gnn_emb.reshape(2 * NV, D)
    ei = edge_index.reshape(2 * NE)
    return _score_kernel(table, ei)


# speedup vs baseline: 9.0473x; 1.5641x over previous
import functools

import jax
import jax.numpy as jnp
from jax import lax
from jax.experimental import pallas as pl
from jax.experimental.pallas import tpu as pltpu
from jax.experimental.pallas import tpu_sc as plsc

D = 128
NV = 10000
NE = 320000

_info = plsc.get_sparse_core_info()
NC, NS, L = _info.num_cores, _info.num_subcores, _info.num_lanes
NW = NC * NS
EPW = NE // NW
C = 80
NCHUNK = EPW // C
G = C // L

_mesh = plsc.VectorSubcoreMesh(core_axis_name="c", subcore_axis_name="s")


@functools.partial(
    pl.kernel,
    mesh=_mesh,
    compiler_params=pltpu.CompilerParams(
        use_tc_tiling_on_sc=False, needs_layout_passes=False
    ),
    out_type=jax.ShapeDtypeStruct((NE,), jnp.float32),
    scratch_types=[
        pltpu.VMEM((EPW,), jnp.int32),
        pltpu.VMEM((EPW,), jnp.int32),
        pltpu.VMEM((C, D), jnp.float32),
        pltpu.VMEM((C, D), jnp.float32),
        pltpu.VMEM((C, D), jnp.float32),
        pltpu.VMEM((C, D), jnp.float32),
        pltpu.VMEM((EPW,), jnp.float32),
        pltpu.SemaphoreType.DMA,
        pltpu.SemaphoreType.DMA,
        pltpu.SemaphoreType.DMA,
        pltpu.SemaphoreType.DMA,
    ],
)
def _score_kernel(table_hbm, ei_hbm, out_hbm, sidx, didx, sa, da, sb, db,
                  sc, sem_sa, sem_da, sem_sb, sem_db):
    wid = lax.axis_index("s") * NC + lax.axis_index("c")
    ebase = wid * EPW
    lane = lax.iota(jnp.int32, L)

    pltpu.sync_copy(ei_hbm.at[pl.ds(ebase, EPW)], sidx)
    pltpu.sync_copy(ei_hbm.at[pl.ds(NE + ebase, EPW)], didx)

    def dbl(i, c):
        sidx[pl.ds(i * L, L)] = sidx[pl.ds(i * L, L)] * 2
        didx[pl.ds(i * L, L)] = didx[pl.ds(i * L, L)] * 2
        return c

    lax.fori_loop(0, EPW // L, dbl, 0, unroll=4)

    def issue(ci, srows, drows, sem_s, sem_d):
        s_cp = pltpu.async_copy(
            table_hbm.at[sidx.at[pl.ds(ci * C, C)]], srows, sem_s)
        d_cp = pltpu.async_copy(
            table_hbm.at[didx.at[pl.ds(ci * C, C)]], drows, sem_d)
        return s_cp, d_cp

    def wait(ci, srows, drows, sem_s, sem_d):
        pltpu.make_async_copy(
            table_hbm.at[sidx.at[pl.ds(ci * C, C)]], srows, sem_s).wait()
        pltpu.make_async_copy(
            table_hbm.at[didx.at[pl.ds(ci * C, C)]], drows, sem_d).wait()

    def compute(ci, srows, drows):
        def grp(g, c):
            rows = g * L + lane

            def dstep(d, acc):
                col = (d + lane) & (D - 1)
                s = plsc.load_gather(srows, [rows, col])
                t = plsc.load_gather(drows, [rows, col])
                return acc + s * t

            acc = lax.fori_loop(0, D, dstep, jnp.zeros((L,), jnp.float32),
                                unroll=8)
            sc[pl.ds(ci * C + g * L, L)] = acc
            return c

        lax.fori_loop(0, G, grp, 0)

    issue(0, sa, da, sem_sa, sem_da)

    def pair(k, carry):
        c0 = 2 * k
        issue(c0 + 1, sb, db, sem_sb, sem_db)
        wait(c0, sa, da, sem_sa, sem_da)
        compute(c0, sa, da)
        issue(c0 + 2, sa, da, sem_sa, sem_da)
        wait(c0 + 1, sb, db, sem_sb, sem_db)
        compute(c0 + 1, sb, db)
        return carry

    lax.fori_loop(0, (NCHUNK - 1) // 2, pair, 0)
    wait(NCHUNK - 1, sa, da, sem_sa, sem_da)
    compute(NCHUNK - 1, sa, da)

    pltpu.sync_copy(sc, out_hbm.at[pl.ds(ebase, EPW)])


def kernel(gnn_emb, edge_index):
    table = gnn_emb.reshape(2 * NV, D)
    ei = edge_index.reshape(2 * NE)
    return _score_kernel(table, ei)


# speedup vs baseline: 10.0870x; 1.1149x over previous
import functools

import jax
import jax.numpy as jnp
from jax import lax
from jax.experimental import pallas as pl
from jax.experimental.pallas import tpu as pltpu
from jax.experimental.pallas import tpu_sc as plsc

D = 128
W = D // 2
NV = 10000
NE = 320000

_info = plsc.get_sparse_core_info()
NC, NS, L = _info.num_cores, _info.num_subcores, _info.num_lanes
NW = NC * NS
EPW = NE // NW
C = 80
NCHUNK = EPW // C
G = C // L

_mesh = plsc.VectorSubcoreMesh(core_axis_name="c", subcore_axis_name="s")


@functools.partial(
    pl.kernel,
    mesh=_mesh,
    compiler_params=pltpu.CompilerParams(
        use_tc_tiling_on_sc=False, needs_layout_passes=False
    ),
    out_type=jax.ShapeDtypeStruct((NE,), jnp.float32),
    scratch_types=[
        pltpu.VMEM((EPW,), jnp.int32),
        pltpu.VMEM((EPW,), jnp.int32),
        pltpu.VMEM((C, W), jnp.int32),
        pltpu.VMEM((C, W), jnp.int32),
        pltpu.VMEM((C, W), jnp.int32),
        pltpu.VMEM((C, W), jnp.int32),
        pltpu.VMEM((EPW,), jnp.float32),
        pltpu.SemaphoreType.DMA,
        pltpu.SemaphoreType.DMA,
        pltpu.SemaphoreType.DMA,
        pltpu.SemaphoreType.DMA,
    ],
)
def _score_kernel(table_hbm, ei_hbm, out_hbm, sidx, didx, sa, da, sb, db,
                  sc, sem_sa, sem_da, sem_sb, sem_db):
    wid = lax.axis_index("s") * NC + lax.axis_index("c")
    ebase = wid * EPW
    lane = lax.iota(jnp.int32, L)

    pltpu.sync_copy(ei_hbm.at[pl.ds(ebase, EPW)], sidx)
    pltpu.sync_copy(ei_hbm.at[pl.ds(NE + ebase, EPW)], didx)

    def issue(ci, srows, drows, sem_s, sem_d):
        pltpu.async_copy(
            table_hbm.at[sidx.at[pl.ds(ci * C, C)]], srows, sem_s)
        pltpu.async_copy(
            table_hbm.at[didx.at[pl.ds(ci * C, C)]], drows, sem_d)

    def wait(ci, srows, drows, sem_s, sem_d):
        pltpu.make_async_copy(
            table_hbm.at[sidx.at[pl.ds(ci * C, C)]], srows, sem_s).wait()
        pltpu.make_async_copy(
            table_hbm.at[didx.at[pl.ds(ci * C, C)]], drows, sem_d).wait()

    def compute(ci, srows, drows):
        def grp(g, c):
            rows = g * L + lane

            def wstep(w, acc):
                col = (w + lane) & (W - 1)
                s32 = plsc.load_gather(srows, [rows, col])
                t32 = plsc.load_gather(drows, [rows, col])
                sbf = plsc.bitcast(s32, jnp.bfloat16)
                tbf = plsc.bitcast(t32, jnp.bfloat16)
                s_a, s_b = plsc.unpack(sbf, format=plsc.PackFormat.INTERLEAVED)
                t_a, t_b = plsc.unpack(tbf, format=plsc.PackFormat.INTERLEAVED)
                return acc + s_a * t_a + s_b * t_b

            acc = lax.fori_loop(0, W, wstep, jnp.zeros((L,), jnp.float32),
                                unroll=8)
            sc[pl.ds(ci * C + g * L, L)] = acc
            return c

        lax.fori_loop(0, G, grp, 0)

    issue(0, sa, da, sem_sa, sem_da)

    def pair(k, carry):
        c0 = 2 * k
        issue(c0 + 1, sb, db, sem_sb, sem_db)
        wait(c0, sa, da, sem_sa, sem_da)
        compute(c0, sa, da)
        issue(c0 + 2, sa, da, sem_sa, sem_da)
        wait(c0 + 1, sb, db, sem_sb, sem_db)
        compute(c0 + 1, sb, db)
        return carry

    lax.fori_loop(0, (NCHUNK - 1) // 2, pair, 0)
    wait(NCHUNK - 1, sa, da, sem_sa, sem_da)
    compute(NCHUNK - 1, sa, da)

    pltpu.sync_copy(sc, out_hbm.at[pl.ds(ebase, EPW)])


def kernel(gnn_emb, edge_index):
    mu16 = gnn_emb[:, :D].astype(jnp.bfloat16)
    table = lax.bitcast_convert_type(mu16.reshape(NV, W, 2), jnp.int32)
    ei = edge_index.reshape(2 * NE)
    return _score_kernel(table, ei)
